# Initial kernel scaffold; baseline (speedup 1.0000x reference)
import jax
import jax.numpy as jnp
from jax.experimental import pallas as pl


def kernel(x, proj_w, proj_b, blk_w1, blk_b1, blk_w2, blk_b2, bn1_g, bn1_b, bn2_g, bn2_b, lin1_w, lin1_b, lin2_w, lin2_b, lin3_w, lin3_b, bnf1_g, bnf1_b, bnf2_g, bnf2_b, edge_index, batch):
    raise NotImplementedError("write your pallas kernel here")


# speedup vs baseline: 14.0643x; 14.0643x over previous
import functools

import jax
import jax.numpy as jnp
from jax import lax
from jax.experimental import pallas as pl
from jax.experimental.pallas import tpu as pltpu
from jax.experimental.pallas import tpu_sc as plsc

_EPS = 1e-5
_NSC = 2


def _bdot(a, b):
    return lax.dot_general(a.astype(jnp.bfloat16), b.astype(jnp.bfloat16),
                           (((1,), (0,)), ((), ())),
                           preferred_element_type=jnp.float32)

_NTILE = 16
_CHUNK = 512
_SUB = 128
_R = 2000


def _init_and_flush(zeros_hbm, acc, out_hbm, s, out_base, n, wb, wb_last):
    def per_tile(op):
        @pl.when(s < _NTILE - 1)
        def _():
            op(s * wb, wb)

        @pl.when(s == _NTILE - 1)
        def _():
            op((_NTILE - 1) * wb, wb_last)

    def zero(r0, sz):
        pltpu.sync_copy(zeros_hbm.at[pl.ds(0, sz)], acc.at[pl.ds(r0, sz)])

    def flush(r0, sz):
        pltpu.sync_copy(acc.at[pl.ds(r0, sz)], out_hbm.at[pl.ds(out_base + r0, sz)])

    return per_tile, zero, flush


def _sc_agg_body(y_hbm, src_hbm, dst_hbm, zeros_hbm, out_hbm,
                 idx_s, idx_d, rows, acc, sem_g, sem_s, *,
                 n, chunks_per_tile, wb, wb_last):
    c = lax.axis_index("c")
    s = lax.axis_index("s")
    per_tile, zero, flush = _init_and_flush(zeros_hbm, acc, out_hbm, s, c * n,
                                            n, wb, wb_last)
    per_tile(zero)
    @pl.when(s == 0)
    def _():
        pltpu.sync_copy(zeros_hbm.at[pl.ds(0, 8)], acc.at[pl.ds(n, 8)])
    plsc.subcore_barrier()

    cn = c * n

    def chunk(k, carry):
        r0 = (s * chunks_per_tile + k) * (_CHUNK // _SUB)
        pltpu.sync_copy(src_hbm.at[pl.ds(r0, _CHUNK // _SUB)], idx_s)
        pltpu.sync_copy(dst_hbm.at[pl.ds(r0, _CHUNK // _SUB)], idx_d)
        for j in range(_CHUNK // _SUB):
            for t in range(_SUB // 16):
                sl = idx_s[j, pl.ds(16 * t, 16)]
                idx_s[j, pl.ds(16 * t, 16)] = sl + cn
        gets = [
            pltpu.async_copy(y_hbm.at[idx_s.at[j]],
                             rows.at[pl.ds(_SUB * j, _SUB)], sem_g)
            for j in range(_CHUNK // _SUB)
        ]
        for d in gets:
            d.wait()
        puts = [
            pltpu.async_copy(rows.at[pl.ds(_SUB * j, _SUB)],
                             acc.at[idx_d.at[j]], sem_s, add=True)
            for j in range(_CHUNK // _SUB)
        ]
        for d in puts:
            d.wait()
        return carry

    lax.fori_loop(0, chunks_per_tile, chunk, 0)
    plsc.subcore_barrier()
    per_tile(flush)


def _sc_deg_body(dst_hbm, zeros_hbm, ones_hbm, out_hbm,
                 idx_d, ones_v, acc, sem_s, *,
                 n, chunks_per_tile, wb, wb_last):
    c = lax.axis_index("c")
    s = lax.axis_index("s")
    per_tile, zero, flush = _init_and_flush(zeros_hbm, acc, out_hbm, s, c * n,
                                            n, wb, wb_last)
    per_tile(zero)
    @pl.when(s == 0)
    def _():
        pltpu.sync_copy(zeros_hbm.at[pl.ds(0, 8)], acc.at[pl.ds(n, 8)])
    pltpu.sync_copy(ones_hbm, ones_v)
    plsc.subcore_barrier()

    w = c * _NTILE + s

    def chunk(k, carry):
        r0 = (w * chunks_per_tile + k) * (_CHUNK // _SUB)
        pltpu.sync_copy(dst_hbm.at[pl.ds(r0, _CHUNK // _SUB)], idx_d)
        puts = [
            pltpu.async_copy(ones_v.at[pl.ds(_SUB * j, _SUB)],
                             acc.at[idx_d.at[j]], sem_s, add=True)
            for j in range(_CHUNK // _SUB)
        ]
        for d in puts:
            d.wait()
        return carry

    lax.fori_loop(0, chunks_per_tile, chunk, 0)
    plsc.subcore_barrier()
    per_tile(flush)


def _dinv_body(d0_ref, d1_ref, dinv_ref):
    deg = d0_ref[0][:, 0:1] + d1_ref[0][:, 0:1] + 1.0
    dinv_ref[...] = jnp.broadcast_to(lax.rsqrt(deg), dinv_ref.shape)


def _proj_body(x_ref, pw_ref, pb_ref, w1_ref, dinv_ref, h_ref, y_ref):
    h = _bdot(x_ref[...], pw_ref[...]) + pb_ref[...]
    h_ref[...] = h
    y = _bdot(h, w1_ref[...]) * dinv_ref[...]
    y_ref[0, :, :] = y[:, :32]
    y_ref[1, :, :] = y[:, 32:]


def _post_body(a0_ref, a1_ref, y0_ref, y1_ref, dinv_ref, b_ref,
               o_ref, st_ref, scr):
    i = pl.program_id(0)
    acc = jnp.concatenate([a0_ref[0], a1_ref[0]], axis=1)
    y = jnp.concatenate([y0_ref[0], y1_ref[0]], axis=1)
    o = dinv_ref[...] * (acc + y) + b_ref[...]
    o_ref[...] = o

    @pl.when(i == 0)
    def _():
        scr[...] = jnp.zeros_like(scr)

    scr[0:1, :] = scr[0:1, :] + jnp.sum(o, axis=0, keepdims=True)
    scr[1:2, :] = scr[1:2, :] + jnp.sum(o * o, axis=0, keepdims=True)

    @pl.when(i == pl.num_programs(0) - 1)
    def _():
        st_ref[...] = scr[...]


def _bn_of(o, st_ref, g_ref, bb_ref, n):
    m = st_ref[0:1, :] / n
    v = st_ref[1:2, :] / n - m * m
    return (o - m) * lax.rsqrt(v + _EPS) * g_ref[...] + bb_ref[...]


def _bnmm_body(o_ref, st_ref, g_ref, bb_ref, w_ref, dinv_ref, y_ref, *, n):
    z = jnp.maximum(_bn_of(o_ref[...], st_ref, g_ref, bb_ref, n), 0.0)
    y = _bdot(z, w_ref[...]) * dinv_ref[...]
    y_ref[0, :, :] = y[:, :32]
    y_ref[1, :, :] = y[:, 32:]


def _resid_mm_body(o_ref, st_ref, g_ref, bb_ref, idn_ref, w_ref, dinv_ref,
                   h_ref, y_ref, *, n):
    h = jnp.maximum(_bn_of(o_ref[...], st_ref, g_ref, bb_ref, n) + idn_ref[...], 0.0)
    h_ref[...] = h
    y = _bdot(h, w_ref[...]) * dinv_ref[...]
    y_ref[0, :, :] = y[:, :32]
    y_ref[1, :, :] = y[:, 32:]


def _resid_last_body(o_ref, st_ref, g_ref, bb_ref, idn_ref, h_ref, *, n):
    h_ref[...] = jnp.maximum(
        _bn_of(o_ref[...], st_ref, g_ref, bb_ref, n) + idn_ref[...], 0.0)


def _pool_body(h_ref, b_ref, z_ref, sums, cnt, mx, *, ng, nblocks):
    i = pl.program_id(0)

    @pl.when(i == 0)
    def _():
        sums[...] = jnp.zeros_like(sums)
        cnt[...] = jnp.zeros_like(cnt)
        mx[...] = jnp.full_like(mx, -jnp.inf)

    b = b_ref[...]
    h = h_ref[...]
    gid = lax.broadcasted_iota(jnp.int32, (1, ng), 1)
    oh = (b == gid).astype(jnp.float32)
    sums[...] = sums[...] + lax.dot_general(
        oh, h, (((0,), (0,)), ((), ())), preferred_element_type=jnp.float32, precision=lax.Precision.HIGHEST)
    cnt[...] = cnt[...] + lax.dot_general(
        oh, jnp.ones_like(h), (((0,), (0,)), ((), ())),
        preferred_element_type=jnp.float32, precision=lax.Precision.HIGHEST)

    def body(g, carry):
        mg = jnp.max(jnp.where(b == g, h, -jnp.inf), axis=0, keepdims=True)
        mx[pl.ds(g, 1), :] = jnp.maximum(mx[pl.ds(g, 1), :], mg)
        return carry

    lax.fori_loop(b[0, 0], b[b.shape[0] - 1, 0] + 1, body, 0)

    @pl.when(i == nblocks - 1)
    def _():
        hdim = mx.shape[1]
        z_ref[:, :hdim] = sums[...] / jnp.maximum(cnt[...], 1.0)
        z_ref[:, hdim:] = mx[...]


def _mlp_body(z_ref, w1_ref, b1_ref, g1_ref, bb1_ref,
              w2_ref, b2_ref, g2_ref, bb2_ref, w3_ref, b3_ref, out_ref):
    def bn(a):
        m = jnp.mean(a, axis=0, keepdims=True)
        v = jnp.mean(a * a, axis=0, keepdims=True) - m * m
        return (a - m) * lax.rsqrt(v + _EPS)

    a = _bdot(z_ref[...], w1_ref[...]) + b1_ref[...]
    a = jnp.maximum(bn(a) * g1_ref[...] + bb1_ref[...], 0.0)
    a = _bdot(a, w2_ref[...]) + b2_ref[...]
    a = jnp.maximum(bn(a) * g2_ref[...] + bb2_ref[...], 0.0)
    out_ref[...] = _bdot(a, w3_ref[...]) + b3_ref[...]


def kernel(x, proj_w, proj_b, blk_w1, blk_b1, blk_w2, blk_b2, bn1_g, bn1_b,
           bn2_g, bn2_b, lin1_w, lin1_b, lin2_w, lin2_b, lin3_w, lin3_b,
           bnf1_g, bnf1_b, bnf2_g, bnf2_b, edge_index, batch):
    n, in_dim = x.shape
    e = edge_index.shape[1]
    hdim = proj_w.shape[1]
    nlayer = blk_w1.shape[0]
    ng = 64
    half = hdim // 2
    f32 = jnp.float32

    nb = n // _R
    assert n % _R == 0 and hdim == 64

    total_chunks = -(-e // _CHUNK)
    total_chunks = -(-total_chunks // (_NSC * _NTILE)) * (_NSC * _NTILE)
    e_p = total_chunks * _CHUNK
    pad = e_p - e
    src = edge_index[0]
    dst = edge_index[1]
    apad = jnp.arange(pad, dtype=jnp.int32)
    src_p = jnp.concatenate([src, apad % n]).reshape(e_p // _SUB, _SUB)
    dst_p = jnp.concatenate([dst, n + (apad % 8)]).reshape(e_p // _SUB, _SUB)

    wb = -(-(n // _NTILE) // 8) * 8
    wb_last = n - (_NTILE - 1) * wb
    assert wb_last > 0 and wb_last % 8 == 0
    zeros32 = jnp.zeros((wb, half), f32)
    zeros16 = jnp.zeros((wb, 16), f32)
    ones16 = jnp.ones((_CHUNK, 16), f32)

    mesh = plsc.VectorSubcoreMesh(core_axis_name="c", subcore_axis_name="s")

    deg_call = pl.kernel(
        functools.partial(_sc_deg_body, n=n,
                          chunks_per_tile=total_chunks // (_NSC * _NTILE),
                          wb=wb, wb_last=wb_last),
        out_type=jax.ShapeDtypeStruct((2 * n, 16), f32),
        mesh=mesh,
        compiler_params=pltpu.CompilerParams(use_tc_tiling_on_sc=False),
        scratch_types=[
            pltpu.VMEM((_CHUNK // _SUB, _SUB), jnp.int32),
            pltpu.VMEM((_CHUNK, 16), f32),
            pltpu.VMEM_SHARED((n + 8, 16), f32),
            pltpu.SemaphoreType.DMA,
        ],
    )
    degp = deg_call(dst_p, zeros16, ones16).reshape(2, n, 16)

    agg_call = pl.kernel(
        functools.partial(_sc_agg_body, n=n,
                          chunks_per_tile=total_chunks // _NTILE,
                          wb=wb, wb_last=wb_last),
        out_type=jax.ShapeDtypeStruct((2 * n, half), f32),
        mesh=mesh,
        compiler_params=pltpu.CompilerParams(use_tc_tiling_on_sc=False),
        scratch_types=[
            pltpu.VMEM((_CHUNK // _SUB, _SUB), jnp.int32),
            pltpu.VMEM((_CHUNK // _SUB, _SUB), jnp.int32),
            pltpu.VMEM((_CHUNK, half), f32),
            pltpu.VMEM_SHARED((n + 8, half), f32),
            pltpu.SemaphoreType.DMA,
            pltpu.SemaphoreType.DMA,
        ],
    )

    def aggregate(y3):
        out = agg_call(y3.reshape(2 * n, half), src_p, dst_p, zeros32)
        return out.reshape(2, n, half)

    row_spec = pl.BlockSpec((_R, hdim), lambda i: (i, 0))
    plane0 = pl.BlockSpec((1, _R, half), lambda i: (0, i, 0))
    plane1 = pl.BlockSpec((1, _R, half), lambda i: (1, i, 0))
    y_out_spec = pl.BlockSpec((2, _R, half), lambda i: (0, i, 0))
    full = lambda shp: pl.BlockSpec(shp, lambda i: tuple(0 for _ in shp))
    st_spec = full((8, hdim))
    vec = lambda a: a.reshape(1, -1)

    y_shape = jax.ShapeDtypeStruct((2, n, half), f32)
    h_shape = jax.ShapeDtypeStruct((n, hdim), f32)
    st_shape = jax.ShapeDtypeStruct((8, hdim), f32)

    dinv = pl.pallas_call(
        _dinv_body, grid=(nb,),
        in_specs=[pl.BlockSpec((1, _R, 16), lambda i: (0, i, 0)),
                  pl.BlockSpec((1, _R, 16), lambda i: (1, i, 0))],
        out_specs=row_spec, out_shape=h_shape,
    )(degp, degp)

    h, y = pl.pallas_call(
        _proj_body, grid=(nb,),
        in_specs=[pl.BlockSpec((_R, in_dim), lambda i: (i, 0)),
                  full((in_dim, hdim)), full((1, hdim)),
                  full((hdim, hdim)), row_spec],
        out_specs=[row_spec, y_out_spec],
        out_shape=[h_shape, y_shape],
    )(x, proj_w, vec(proj_b), blk_w1[0], dinv)

    post_call = pl.pallas_call(
        _post_body, grid=(nb,),
        in_specs=[plane0, plane1, plane0, plane1, row_spec, full((1, hdim))],
        out_specs=[row_spec, st_spec],
        out_shape=[h_shape, st_shape],
        scratch_shapes=[pltpu.VMEM((8, hdim), f32)],
    )

    bnmm_call = pl.pallas_call(
        functools.partial(_bnmm_body, n=n), grid=(nb,),
        in_specs=[row_spec, st_spec, full((1, hdim)), full((1, hdim)),
                  full((hdim, hdim)), row_spec],
        out_specs=y_out_spec, out_shape=y_shape,
    )

    resid_mm_call = pl.pallas_call(
        functools.partial(_resid_mm_body, n=n), grid=(nb,),
        in_specs=[row_spec, st_spec, full((1, hdim)), full((1, hdim)),
                  row_spec, full((hdim, hdim)), row_spec],
        out_specs=[row_spec, y_out_spec],
        out_shape=[h_shape, y_shape],
    )

    resid_last_call = pl.pallas_call(
        functools.partial(_resid_last_body, n=n), grid=(nb,),
        in_specs=[row_spec, st_spec, full((1, hdim)), full((1, hdim)), row_spec],
        out_specs=row_spec, out_shape=h_shape,
    )

    for i in range(nlayer):
        acc1 = aggregate(y)
        o1, st1 = post_call(acc1, acc1, y, y, dinv, vec(blk_b1[i]))
        y2 = bnmm_call(o1, st1, vec(bn1_g[i]), vec(bn1_b[i]), blk_w2[i], dinv)
        acc2 = aggregate(y2)
        o2, st2 = post_call(acc2, acc2, y2, y2, dinv, vec(blk_b2[i]))
        if i + 1 < nlayer:
            h, y = resid_mm_call(o2, st2, vec(bn2_g[i]), vec(bn2_b[i]), h,
                                 blk_w1[i + 1], dinv)
        else:
            h = resid_last_call(o2, st2, vec(bn2_g[i]), vec(bn2_b[i]), h)

    batch2 = batch.reshape(n, 1)
    z = pl.pallas_call(
        functools.partial(_pool_body, ng=ng, nblocks=nb), grid=(nb,),
        in_specs=[row_spec, pl.BlockSpec((_R, 1), lambda i: (i, 0))],
        out_specs=full((ng, 2 * hdim)),
        out_shape=jax.ShapeDtypeStruct((ng, 2 * hdim), f32),
        scratch_shapes=[pltpu.VMEM((ng, hdim), f32),
                        pltpu.VMEM((ng, hdim), f32),
                        pltpu.VMEM((ng, hdim), f32)],
    )(h, batch2)

    out = pl.pallas_call(
        _mlp_body,
        in_specs=[
            pl.BlockSpec((ng, 2 * hdim), lambda: (0, 0)),
            pl.BlockSpec((2 * hdim, hdim), lambda: (0, 0)),
            pl.BlockSpec((1, hdim), lambda: (0, 0)),
            pl.BlockSpec((1, hdim), lambda: (0, 0)),
            pl.BlockSpec((1, hdim), lambda: (0, 0)),
            pl.BlockSpec((hdim, half), lambda: (0, 0)),
            pl.BlockSpec((1, half), lambda: (0, 0)),
            pl.BlockSpec((1, half), lambda: (0, 0)),
            pl.BlockSpec((1, half), lambda: (0, 0)),
            pl.BlockSpec((half, 2), lambda: (0, 0)),
            pl.BlockSpec((1, 2), lambda: (0, 0)),
        ],
        out_specs=pl.BlockSpec((ng, 2), lambda: (0, 0)),
        out_shape=jax.ShapeDtypeStruct((ng, 2), f32),
    )(z, lin1_w, vec(lin1_b), vec(bnf1_g), vec(bnf1_b),
      lin2_w, vec(lin2_b), vec(bnf2_g), vec(bnf2_b), lin3_w, vec(lin3_b))

    return out
